# baseline (speedup 1.0000x reference)
import functools

import jax
import jax.numpy as jnp
from jax import lax
from jax.experimental import pallas as pl
from jax.experimental.pallas import tpu as pltpu
from jax.experimental.pallas import tpu_sc as plsc

D = 64
DP = 128
B_TOTAL = 69632 * 5
NC, NS = 2, 16
NW = NC * NS
CHUNK = 320
GROUPS = 34
B_PER_W = CHUNK * GROUPS
assert B_PER_W * NW == B_TOTAL

_mesh = plsc.VectorSubcoreMesh(core_axis_name="c", subcore_axis_name="s")

V_ROWS = 1000001
VP_ROWS = 1000008
TBLK = 4096


def _transpose_pad_block(tt_ref, out_ref):
    x = tt_ref[...]
    row = jax.lax.broadcasted_iota(jnp.int32, (D, D), 0)
    col = jax.lax.broadcasted_iota(jnp.int32, (D, D), 1)
    eye = (row == col).astype(jnp.float32)
    xt = jax.lax.dot_general(
        x,
        eye,
        (((0,), (0,)), ((), ())),
        preferred_element_type=jnp.float32,
        precision=jax.lax.Precision.HIGHEST,
    )
    out_ref[...] = jnp.concatenate([xt, xt], axis=1)


def _transpose_pad(tt):
    grid = (VP_ROWS + TBLK - 1) // TBLK
    return pl.pallas_call(
        _transpose_pad_block,
        grid=(grid,),
        in_specs=[pl.BlockSpec((D, TBLK), lambda i: (0, i))],
        out_specs=pl.BlockSpec((TBLK, DP), lambda i: (i, 0)),
        out_shape=jax.ShapeDtypeStruct((VP_ROWS, DP), jnp.float32),
    )(tt)


@functools.partial(
    pl.kernel,
    mesh=_mesh,
    out_type=jax.ShapeDtypeStruct((B_TOTAL, DP), jnp.float32),
    scratch_types=[
        pltpu.VMEM((B_PER_W,), jnp.int32),
        pltpu.VMEM((2, CHUNK, DP), jnp.float32),
        pltpu.SemaphoreType.DMA,
        pltpu.SemaphoreType.DMA,
        pltpu.SemaphoreType.DMA,
        pltpu.SemaphoreType.DMA,
    ],
    compiler_params=pltpu.CompilerParams(use_tc_tiling_on_sc=False),
)
def _gather_kernel(table_hbm, idx_hbm, out_hbm, idx_v, rows_v, sg0, sg1, ss0, ss1):
    wid = lax.axis_index("s") * NC + lax.axis_index("c")
    pltpu.sync_copy(idx_hbm.at[pl.ds(wid * B_PER_W, B_PER_W)], idx_v)
    base = wid * B_PER_W
    sem_g = (sg0, sg1)
    sem_s = (ss0, ss1)

    g_copies = [None] * GROUPS
    s_copies = [None] * GROUPS
    for g in range(GROUPS):
        b = g % 2
        if g >= 2:
            s_copies[g - 2].wait()
        g_copies[g] = pltpu.async_copy(
            table_hbm.at[idx_v.at[pl.ds(g * CHUNK, CHUNK)]], rows_v.at[b], sem_g[b]
        )
        if g >= 1:
            pb = (g - 1) % 2
            g_copies[g - 1].wait()
            s_copies[g - 1] = pltpu.async_copy(
                rows_v.at[pb, :, pl.ds(0, D)],
                out_hbm.at[pl.ds(base + (g - 1) * CHUNK, CHUNK), pl.ds(0, D)],
                sem_s[pb],
            )
    lb = (GROUPS - 1) % 2
    g_copies[GROUPS - 1].wait()
    s_copies[GROUPS - 1] = pltpu.async_copy(
        rows_v.at[lb, :, pl.ds(0, D)],
        out_hbm.at[pl.ds(base + (GROUPS - 1) * CHUNK, CHUNK), pl.ds(0, D)],
        sem_s[lb],
    )
    s_copies[GROUPS - 2].wait()
    s_copies[GROUPS - 1].wait()


def kernel(values, table):
    tpad = _transpose_pad(table.T)
    idx = values.reshape(-1)
    out_pad = _gather_kernel(tpad, idx)
    return out_pad[:, :D]


# speedup vs baseline: 1.4287x; 1.4287x over previous
import functools

import jax
import jax.numpy as jnp
from jax import lax
from jax.experimental import pallas as pl
from jax.experimental.pallas import tpu as pltpu
from jax.experimental.pallas import tpu_sc as plsc

D = 64
DP = 128
B_TOTAL = 69632 * 5
NC, NS = 2, 16
NW = NC * NS
CHUNK = 320
GROUPS = 34
B_PER_W = CHUNK * GROUPS
assert B_PER_W * NW == B_TOTAL

_mesh = plsc.VectorSubcoreMesh(core_axis_name="c", subcore_axis_name="s")

V_ROWS = 1000001
VP_ROWS = 1000008
TBLK = 24576


def _transpose_pad_block(tt_ref, out_ref):
    x = tt_ref[...]
    xt = jnp.swapaxes(x, 0, 1)
    out_ref[...] = jnp.concatenate([xt, xt], axis=1)


def _transpose_pad(tt):
    grid = (VP_ROWS + TBLK - 1) // TBLK
    return pl.pallas_call(
        _transpose_pad_block,
        grid=(grid,),
        in_specs=[pl.BlockSpec((D, TBLK), lambda i: (0, i))],
        out_specs=pl.BlockSpec((TBLK, DP), lambda i: (i, 0)),
        out_shape=jax.ShapeDtypeStruct((VP_ROWS, DP), jnp.float32),
    )(tt)


@functools.partial(
    pl.kernel,
    mesh=_mesh,
    out_type=jax.ShapeDtypeStruct((B_TOTAL, DP), jnp.float32),
    scratch_types=[
        pltpu.VMEM((B_PER_W,), jnp.int32),
        pltpu.VMEM((2, CHUNK, DP), jnp.float32),
        pltpu.SemaphoreType.DMA,
        pltpu.SemaphoreType.DMA,
        pltpu.SemaphoreType.DMA,
        pltpu.SemaphoreType.DMA,
    ],
    compiler_params=pltpu.CompilerParams(use_tc_tiling_on_sc=False),
)
def _gather_kernel(table_hbm, idx_hbm, out_hbm, idx_v, rows_v, sg0, sg1, ss0, ss1):
    wid = lax.axis_index("s") * NC + lax.axis_index("c")
    pltpu.sync_copy(idx_hbm.at[pl.ds(wid * B_PER_W, B_PER_W)], idx_v)
    base = wid * B_PER_W
    sem_g = (sg0, sg1)
    sem_s = (ss0, ss1)

    g_copies = [None] * GROUPS
    s_copies = [None] * GROUPS
    for g in range(GROUPS):
        b = g % 2
        if g >= 2:
            s_copies[g - 2].wait()
        g_copies[g] = pltpu.async_copy(
            table_hbm.at[idx_v.at[pl.ds(g * CHUNK, CHUNK)]], rows_v.at[b], sem_g[b]
        )
        if g >= 1:
            pb = (g - 1) % 2
            g_copies[g - 1].wait()
            s_copies[g - 1] = pltpu.async_copy(
                rows_v.at[pb, :, pl.ds(0, D)],
                out_hbm.at[pl.ds(base + (g - 1) * CHUNK, CHUNK), pl.ds(0, D)],
                sem_s[pb],
            )
    lb = (GROUPS - 1) % 2
    g_copies[GROUPS - 1].wait()
    s_copies[GROUPS - 1] = pltpu.async_copy(
        rows_v.at[lb, :, pl.ds(0, D)],
        out_hbm.at[pl.ds(base + (GROUPS - 1) * CHUNK, CHUNK), pl.ds(0, D)],
        sem_s[lb],
    )
    s_copies[GROUPS - 2].wait()
    s_copies[GROUPS - 1].wait()


def kernel(values, table):
    tpad = _transpose_pad(table.T)
    idx = values.reshape(-1)
    out_pad = _gather_kernel(tpad, idx)
    return out_pad[:, :D]
